# baseline (speedup 1.0000x reference)
import jax
import jax.numpy as jnp
from jax.experimental import pallas as pl
from jax.experimental.pallas import tpu as pltpu

_N, _H, _NH, _DH = 2048, 512, 8, 64
_BA = 256
_BQ = 256
_NP = _N // _BA
_SCALE = 0.125 * 1.4426950408889634


def _body(px_ref, sx_ref, pxT_ref, sxT_ref, Ws_ref, bs_ref, g_ref, b_ref,
          Wp_ref, Wq_ref, Wk_ref, Wv_ref, Wo_ref, nc_ref, nr_ref,
          o_ref,
          hs, qs, ks, vs, bias_s, fqs, gks):
    i = pl.program_id(0)

    @pl.when(i < _NP)
    def _embed():
        rows = pl.ds(i * _BA, _BA)
        px = px_ref[...]
        sx = sx_ref[...]
        h = jax.lax.dot_general(sx, Ws_ref[...], (((1,), (0,)), ((), ())),
                                preferred_element_type=jnp.float32)
        h = h + bs_ref[...]
        mu = jnp.mean(h, axis=1, keepdims=True)
        var = jnp.mean((h - mu) ** 2, axis=1, keepdims=True)
        h = (h - mu) / jnp.sqrt(var + 1e-5) * g_ref[...] + b_ref[...]
        h = h * jax.nn.sigmoid(h)
        h = h + jax.lax.dot_general(px, Wp_ref[...], (((1,), (0,)), ((), ())),
                                    preferred_element_type=jnp.float32)
        hs[rows, :] = h
        d = (px[:, 6:7] > 0).astype(jnp.int32)
        a = (px[:, 7:8] > 0).astype(jnp.int32)
        ar = (sx[:, 1:2] > 0).astype(jnp.int32)
        fqs[rows, :] = d + 2 * a + 4 * ar
        dr = (pxT_ref[6:7, :] > 0).astype(jnp.int32)
        ac = (pxT_ref[7:8, :] > 0).astype(jnp.int32)
        arr = (sxT_ref[1:2, :] > 0).astype(jnp.int32)
        gks[...] = 2 * dr + ac + 4 * arr
        mask = ((fqs[rows, :] & gks[...]) != 0) & (nc_ref[...] != nr_ref[...])
        anyf = jnp.any(mask, axis=1, keepdims=True).astype(jnp.float32)
        bias_s[rows, :] = jnp.where(
            mask, 1.0, 1.0 - anyf).astype(jnp.bfloat16)
        hb = h.astype(jnp.bfloat16)
        qs[rows, :] = (jax.lax.dot_general(
            hb, Wq_ref[...], (((1,), (0,)), ((), ())),
            preferred_element_type=jnp.float32)
            * (_SCALE * anyf)).astype(jnp.bfloat16)
        kblk = jax.lax.dot_general(
            hb, Wk_ref[...], (((1,), (0,)), ((), ())),
            preferred_element_type=jnp.float32).astype(jnp.bfloat16)
        ks[:, rows] = kblk.T
        vblk = jax.lax.dot_general(
            hb, Wv_ref[...], (((1,), (0,)), ((), ())),
            preferred_element_type=jnp.float32).astype(jnp.bfloat16)
        ones = jnp.ones((_BA, 1), jnp.bfloat16)
        zpad = jnp.zeros((_BA, 128 - _DH - 1), jnp.bfloat16)
        for hh in range(_NH):
            sl = slice(hh * _DH, (hh + 1) * _DH)
            vs[rows, hh * 128:(hh + 1) * 128] = jnp.concatenate(
                [vblk[:, sl], ones, zpad], axis=1)

    @pl.when(i >= _NP)
    def _attn():
        j = i - _NP
        rows = pl.ds(j * _BQ, _BQ)
        q = qs[rows, :]
        v = vs[...]
        parts = []
        for hh in range(_NH):
            sl = slice(hh * _DH, (hh + 1) * _DH)
            s = jax.lax.dot_general(q[:, sl], ks[sl, :],
                                    (((1,), (0,)), ((), ())),
                                    preferred_element_type=jnp.float32)
            p = jnp.exp2(s).astype(jnp.bfloat16) * bias_s[rows, :]
            pv = jax.lax.dot_general(
                p, v[:, hh * 128:(hh + 1) * 128], (((1,), (0,)), ((), ())),
                preferred_element_type=jnp.float32)
            parts.append(pv[:, :_DH] * (1.0 / pv[:, _DH:_DH + 1]))
        ctx_all = jnp.concatenate(parts, axis=1).astype(jnp.bfloat16)
        o_ref[...] = hs[rows, :] + jax.lax.dot_general(
            ctx_all, Wo_ref[...], (((1,), (0,)), ((), ())),
            preferred_element_type=jnp.float32)


def kernel(physics_x, structural_x, W_struct, b_struct, gamma, beta,
           W_phys, Wq, Wk, Wv, Wo, atom_to_nuc):
    nuc_col = atom_to_nuc.astype(jnp.int32).reshape(_N, 1)
    nuc_row = atom_to_nuc.astype(jnp.int32).reshape(1, _N)

    def _lo(i):
        return jnp.minimum(i, _NP - 1)

    def _hi(i):
        return jnp.maximum(i - _NP, 0)

    out = pl.pallas_call(
        _body,
        grid=(2 * _NP,),
        in_specs=[
            pl.BlockSpec((_BA, 10), lambda i: (_lo(i), 0)),
            pl.BlockSpec((_BA, 4), lambda i: (_lo(i), 0)),
            pl.BlockSpec((10, _N), lambda i: (0, 0)),
            pl.BlockSpec((4, _N), lambda i: (0, 0)),
            pl.BlockSpec((4, _H), lambda i: (0, 0)),
            pl.BlockSpec((1, _H), lambda i: (0, 0)),
            pl.BlockSpec((1, _H), lambda i: (0, 0)),
            pl.BlockSpec((1, _H), lambda i: (0, 0)),
            pl.BlockSpec((10, _H), lambda i: (0, 0)),
            pl.BlockSpec((_H, _H), lambda i: (0, 0)),
            pl.BlockSpec((_H, _H), lambda i: (0, 0)),
            pl.BlockSpec((_H, _H), lambda i: (0, 0)),
            pl.BlockSpec((_H, _H), lambda i: (0, 0)),
            pl.BlockSpec((_BA, 1), lambda i: (_lo(i), 0)),
            pl.BlockSpec((1, _N), lambda i: (0, 0)),
        ],
        out_specs=pl.BlockSpec((_BQ, _H), lambda i: (_hi(i), 0)),
        out_shape=jax.ShapeDtypeStruct((_N, _H), jnp.float32),
        scratch_shapes=[
            pltpu.VMEM((_N, _H), jnp.float32),
            pltpu.VMEM((_N, _H), jnp.bfloat16),
            pltpu.VMEM((_H, _N), jnp.bfloat16),
            pltpu.VMEM((_N, _NH * 128), jnp.bfloat16),
            pltpu.VMEM((_N, _N), jnp.bfloat16),
            pltpu.VMEM((_N, 1), jnp.int32),
            pltpu.VMEM((1, _N), jnp.int32),
        ],
    )(physics_x, structural_x, physics_x.T, structural_x.T,
      W_struct, b_struct.reshape(1, _H), gamma.reshape(1, _H),
      beta.reshape(1, _H), W_phys, Wq.astype(jnp.bfloat16),
      Wk.astype(jnp.bfloat16), Wv.astype(jnp.bfloat16),
      Wo.astype(jnp.bfloat16), nuc_col, nuc_row)
    return out


# speedup vs baseline: 1.0014x; 1.0014x over previous
import jax
import jax.numpy as jnp
from jax.experimental import pallas as pl
from jax.experimental.pallas import tpu as pltpu

_N, _H, _NH, _DH = 2048, 512, 8, 64
_BA = 256
_BQ = 256
_NP = _N // _BA
_SCALE = 0.125 * 1.4426950408889634


def _body(px_ref, sx_ref, pxT_ref, sxT_ref, Ws_ref, bs_ref, g_ref, b_ref,
          Wp_ref, Wq_ref, Wk_ref, Wv_ref, Wo_ref, nc_ref, nr_ref,
          o_ref,
          hs, qs, ks, vs, bias_s, fqs, gks):
    i = pl.program_id(0)

    @pl.when(i < _NP)
    def _embed():
        rows = pl.ds(i * _BA, _BA)
        px = px_ref[...]
        sx = sx_ref[...]
        h = jax.lax.dot_general(sx, Ws_ref[...], (((1,), (0,)), ((), ())),
                                preferred_element_type=jnp.float32)
        h = h + bs_ref[...]
        mu = jnp.mean(h, axis=1, keepdims=True)
        var = jnp.mean((h - mu) ** 2, axis=1, keepdims=True)
        h = (h - mu) / jnp.sqrt(var + 1e-5) * g_ref[...] + b_ref[...]
        h = h * jax.nn.sigmoid(h)
        h = h + jax.lax.dot_general(px, Wp_ref[...], (((1,), (0,)), ((), ())),
                                    preferred_element_type=jnp.float32)
        hs[rows, :] = h
        d = (px[:, 6:7] > 0).astype(jnp.int32)
        a = (px[:, 7:8] > 0).astype(jnp.int32)
        ar = (sx[:, 1:2] > 0).astype(jnp.int32)
        fqs[rows, :] = d + 2 * a + 4 * ar
        dr = (pxT_ref[6:7, :] > 0).astype(jnp.int32)
        ac = (pxT_ref[7:8, :] > 0).astype(jnp.int32)
        arr = (sxT_ref[1:2, :] > 0).astype(jnp.int32)
        gks[...] = 2 * dr + ac + 4 * arr
        mask = ((fqs[rows, :] & gks[...]) != 0) & (nc_ref[...] != nr_ref[...])
        anyf = jnp.any(mask, axis=1, keepdims=True).astype(jnp.float32)
        bias_s[rows, :] = jnp.where(
            mask, 0.0, -30000.0 * anyf).astype(jnp.bfloat16)
        hb = h.astype(jnp.bfloat16)
        qs[rows, :] = (jax.lax.dot_general(
            hb, Wq_ref[...], (((1,), (0,)), ((), ())),
            preferred_element_type=jnp.float32)
            * (_SCALE * anyf)).astype(jnp.bfloat16)
        kblk = jax.lax.dot_general(
            hb, Wk_ref[...], (((1,), (0,)), ((), ())),
            preferred_element_type=jnp.float32).astype(jnp.bfloat16)
        ks[:, rows] = kblk.T
        vblk = jax.lax.dot_general(
            hb, Wv_ref[...], (((1,), (0,)), ((), ())),
            preferred_element_type=jnp.float32).astype(jnp.bfloat16)
        ones = jnp.ones((_BA, 1), jnp.bfloat16)
        zpad = jnp.zeros((_BA, 128 - _DH - 1), jnp.bfloat16)
        for hh in range(_NH):
            sl = slice(hh * _DH, (hh + 1) * _DH)
            vs[rows, hh * 128:(hh + 1) * 128] = jnp.concatenate(
                [vblk[:, sl], ones, zpad], axis=1)

    @pl.when(i >= _NP)
    def _attn():
        j = i - _NP
        rows = pl.ds(j * _BQ, _BQ)
        q = qs[rows, :]
        v = vs[...]
        parts = []
        for hh in range(_NH):
            sl = slice(hh * _DH, (hh + 1) * _DH)
            s = jax.lax.dot_general(q[:, sl], ks[sl, :],
                                    (((1,), (0,)), ((), ())),
                                    preferred_element_type=jnp.float32)
            p = jnp.exp2(s + bias_s[rows, :].astype(jnp.float32)
                         ).astype(jnp.bfloat16)
            pv = jax.lax.dot_general(
                p, v[:, hh * 128:(hh + 1) * 128], (((1,), (0,)), ((), ())),
                preferred_element_type=jnp.float32)
            parts.append(pv[:, :_DH] * (1.0 / pv[:, _DH:_DH + 1]))
        ctx_all = jnp.concatenate(parts, axis=1).astype(jnp.bfloat16)
        o_ref[...] = hs[rows, :] + jax.lax.dot_general(
            ctx_all, Wo_ref[...], (((1,), (0,)), ((), ())),
            preferred_element_type=jnp.float32)


def kernel(physics_x, structural_x, W_struct, b_struct, gamma, beta,
           W_phys, Wq, Wk, Wv, Wo, atom_to_nuc):
    nuc_col = atom_to_nuc.astype(jnp.int32).reshape(_N, 1)
    nuc_row = atom_to_nuc.astype(jnp.int32).reshape(1, _N)

    def _lo(i):
        return jnp.minimum(i, _NP - 1)

    def _hi(i):
        return jnp.maximum(i - _NP, 0)

    out = pl.pallas_call(
        _body,
        grid=(2 * _NP,),
        in_specs=[
            pl.BlockSpec((_BA, 10), lambda i: (_lo(i), 0)),
            pl.BlockSpec((_BA, 4), lambda i: (_lo(i), 0)),
            pl.BlockSpec((10, _N), lambda i: (0, 0)),
            pl.BlockSpec((4, _N), lambda i: (0, 0)),
            pl.BlockSpec((4, _H), lambda i: (0, 0)),
            pl.BlockSpec((1, _H), lambda i: (0, 0)),
            pl.BlockSpec((1, _H), lambda i: (0, 0)),
            pl.BlockSpec((1, _H), lambda i: (0, 0)),
            pl.BlockSpec((10, _H), lambda i: (0, 0)),
            pl.BlockSpec((_H, _H), lambda i: (0, 0)),
            pl.BlockSpec((_H, _H), lambda i: (0, 0)),
            pl.BlockSpec((_H, _H), lambda i: (0, 0)),
            pl.BlockSpec((_H, _H), lambda i: (0, 0)),
            pl.BlockSpec((_BA, 1), lambda i: (_lo(i), 0)),
            pl.BlockSpec((1, _N), lambda i: (0, 0)),
        ],
        out_specs=pl.BlockSpec((_BQ, _H), lambda i: (_hi(i), 0)),
        out_shape=jax.ShapeDtypeStruct((_N, _H), jnp.float32),
        scratch_shapes=[
            pltpu.VMEM((_N, _H), jnp.float32),
            pltpu.VMEM((_N, _H), jnp.bfloat16),
            pltpu.VMEM((_H, _N), jnp.bfloat16),
            pltpu.VMEM((_N, _NH * 128), jnp.bfloat16),
            pltpu.VMEM((_N, _N), jnp.bfloat16),
            pltpu.VMEM((_N, 1), jnp.int32),
            pltpu.VMEM((1, _N), jnp.int32),
        ],
    )(physics_x, structural_x, physics_x.T, structural_x.T,
      W_struct, b_struct.reshape(1, _H), gamma.reshape(1, _H),
      beta.reshape(1, _H), W_phys, Wq.astype(jnp.bfloat16),
      Wk.astype(jnp.bfloat16), Wv.astype(jnp.bfloat16),
      Wo.astype(jnp.bfloat16), nuc_col, nuc_row)
    return out


# speedup vs baseline: 1.1036x; 1.1021x over previous
import jax
import jax.numpy as jnp
from jax.experimental import pallas as pl
from jax.experimental.pallas import tpu as pltpu

_N, _H, _NH, _DH = 2048, 512, 8, 64
_BA = 256
_BQ = 512
_NP = _N // _BA
_NQ = _N // _BQ
_SCALE = 0.125 * 1.4426950408889634


def _body(px_ref, sx_ref, pxT_ref, sxT_ref, Ws_ref, bs_ref, g_ref, b_ref,
          Wp_ref, Wq_ref, Wk_ref, Wv_ref, Wo_ref, nc_ref, nr_ref,
          o_ref,
          hs, qs, ks, vs, bias_s, fqs, gks):
    i = pl.program_id(0)

    @pl.when(i < _NP)
    def _embed():
        rows = pl.ds(i * _BA, _BA)
        px = px_ref[...]
        sx = sx_ref[...]
        h = jax.lax.dot_general(sx, Ws_ref[...], (((1,), (0,)), ((), ())),
                                preferred_element_type=jnp.float32)
        h = h + bs_ref[...]
        mu = jnp.mean(h, axis=1, keepdims=True)
        var = jnp.mean((h - mu) ** 2, axis=1, keepdims=True)
        h = (h - mu) / jnp.sqrt(var + 1e-5) * g_ref[...] + b_ref[...]
        h = h * jax.nn.sigmoid(h)
        h = h + jax.lax.dot_general(px, Wp_ref[...], (((1,), (0,)), ((), ())),
                                    preferred_element_type=jnp.float32)
        hs[rows, :] = h
        d = (px[:, 6:7] > 0).astype(jnp.int32)
        a = (px[:, 7:8] > 0).astype(jnp.int32)
        ar = (sx[:, 1:2] > 0).astype(jnp.int32)
        fqs[rows, :] = d + 2 * a + 4 * ar
        dr = (pxT_ref[6:7, :] > 0).astype(jnp.int32)
        ac = (pxT_ref[7:8, :] > 0).astype(jnp.int32)
        arr = (sxT_ref[1:2, :] > 0).astype(jnp.int32)
        gks[...] = 2 * dr + ac + 4 * arr
        mask = ((fqs[rows, :] & gks[...]) != 0) & (nc_ref[...] != nr_ref[...])
        anyf = jnp.any(mask, axis=1, keepdims=True).astype(jnp.float32)
        bias_s[rows, :] = jnp.where(
            mask, 0.0, -30000.0 * anyf).astype(jnp.bfloat16)
        hb = h.astype(jnp.bfloat16)
        qs[rows, :] = (jax.lax.dot_general(
            hb, Wq_ref[...], (((1,), (0,)), ((), ())),
            preferred_element_type=jnp.float32)
            * (_SCALE * anyf)).astype(jnp.bfloat16)
        kblk = jax.lax.dot_general(
            hb, Wk_ref[...], (((1,), (0,)), ((), ())),
            preferred_element_type=jnp.float32).astype(jnp.bfloat16)
        ks[:, rows] = kblk.T
        vblk = jax.lax.dot_general(
            hb, Wv_ref[...], (((1,), (0,)), ((), ())),
            preferred_element_type=jnp.float32).astype(jnp.bfloat16)
        ones = jnp.ones((_BA, 1), jnp.bfloat16)
        zpad = jnp.zeros((_BA, 128 - _DH - 1), jnp.bfloat16)
        for hh in range(_NH):
            sl = slice(hh * _DH, (hh + 1) * _DH)
            vs[rows, hh * 128:(hh + 1) * 128] = jnp.concatenate(
                [vblk[:, sl], ones, zpad], axis=1)

    @pl.when(i >= _NP)
    def _attn():
        j = i - _NP
        rows = pl.ds(j * _BQ, _BQ)
        q = qs[rows, :]
        v = vs[...]
        parts = []
        for hh in range(_NH):
            sl = slice(hh * _DH, (hh + 1) * _DH)
            s = jax.lax.dot_general(q[:, sl], ks[sl, :],
                                    (((1,), (0,)), ((), ())),
                                    preferred_element_type=jnp.float32)
            p = jnp.exp2(s + bias_s[rows, :].astype(jnp.float32)
                         ).astype(jnp.bfloat16)
            pv = jax.lax.dot_general(
                p, v[:, hh * 128:(hh + 1) * 128], (((1,), (0,)), ((), ())),
                preferred_element_type=jnp.float32)
            parts.append(pv[:, :_DH] / pv[:, _DH:_DH + 1])
        ctx_all = jnp.concatenate(parts, axis=1).astype(jnp.bfloat16)
        o_ref[...] = hs[rows, :] + jax.lax.dot_general(
            ctx_all, Wo_ref[...], (((1,), (0,)), ((), ())),
            preferred_element_type=jnp.float32)


def kernel(physics_x, structural_x, W_struct, b_struct, gamma, beta,
           W_phys, Wq, Wk, Wv, Wo, atom_to_nuc):
    nuc_col = atom_to_nuc.astype(jnp.int32).reshape(_N, 1)
    nuc_row = atom_to_nuc.astype(jnp.int32).reshape(1, _N)

    def _lo(i):
        return jnp.minimum(i, _NP - 1)

    def _hi(i):
        return jnp.maximum(i - _NP, 0)

    out = pl.pallas_call(
        _body,
        grid=(_NP + _NQ,),
        in_specs=[
            pl.BlockSpec((_BA, 10), lambda i: (_lo(i), 0)),
            pl.BlockSpec((_BA, 4), lambda i: (_lo(i), 0)),
            pl.BlockSpec((10, _N), lambda i: (0, 0)),
            pl.BlockSpec((4, _N), lambda i: (0, 0)),
            pl.BlockSpec((4, _H), lambda i: (0, 0)),
            pl.BlockSpec((1, _H), lambda i: (0, 0)),
            pl.BlockSpec((1, _H), lambda i: (0, 0)),
            pl.BlockSpec((1, _H), lambda i: (0, 0)),
            pl.BlockSpec((10, _H), lambda i: (0, 0)),
            pl.BlockSpec((_H, _H), lambda i: (0, 0)),
            pl.BlockSpec((_H, _H), lambda i: (0, 0)),
            pl.BlockSpec((_H, _H), lambda i: (0, 0)),
            pl.BlockSpec((_H, _H), lambda i: (0, 0)),
            pl.BlockSpec((_BA, 1), lambda i: (_lo(i), 0)),
            pl.BlockSpec((1, _N), lambda i: (0, 0)),
        ],
        out_specs=pl.BlockSpec((_BQ, _H), lambda i: (_hi(i), 0)),
        out_shape=jax.ShapeDtypeStruct((_N, _H), jnp.float32),
        scratch_shapes=[
            pltpu.VMEM((_N, _H), jnp.float32),
            pltpu.VMEM((_N, _H), jnp.bfloat16),
            pltpu.VMEM((_H, _N), jnp.bfloat16),
            pltpu.VMEM((_N, _NH * 128), jnp.bfloat16),
            pltpu.VMEM((_N, _N), jnp.bfloat16),
            pltpu.VMEM((_N, 1), jnp.int32),
            pltpu.VMEM((1, _N), jnp.int32),
        ],
    )(physics_x, structural_x, physics_x.T, structural_x.T,
      W_struct, b_struct.reshape(1, _H), gamma.reshape(1, _H),
      beta.reshape(1, _H), W_phys, Wq.astype(jnp.bfloat16),
      Wk.astype(jnp.bfloat16), Wv.astype(jnp.bfloat16),
      Wo.astype(jnp.bfloat16), nuc_col, nuc_row)
    return out


# speedup vs baseline: 1.1120x; 1.0077x over previous
import jax
import jax.numpy as jnp
from jax.experimental import pallas as pl
from jax.experimental.pallas import tpu as pltpu

_N, _H, _NH, _DH = 2048, 512, 8, 64
_BA = 256
_BQ = 1024
_NP = _N // _BA
_NQ = _N // _BQ
_SCALE = 0.125 * 1.4426950408889634


def _body(px_ref, sx_ref, pxT_ref, sxT_ref, Ws_ref, bs_ref, g_ref, b_ref,
          Wp_ref, Wq_ref, Wk_ref, Wv_ref, Wo_ref, nc_ref, nr_ref,
          o_ref,
          hs, qs, ks, vs, bias_s, fqs, gks):
    i = pl.program_id(0)

    @pl.when(i < _NP)
    def _embed():
        rows = pl.ds(i * _BA, _BA)
        px = px_ref[...]
        sx = sx_ref[...]
        h = jax.lax.dot_general(sx, Ws_ref[...], (((1,), (0,)), ((), ())),
                                preferred_element_type=jnp.float32)
        h = h + bs_ref[...]
        mu = jnp.mean(h, axis=1, keepdims=True)
        var = jnp.mean((h - mu) ** 2, axis=1, keepdims=True)
        h = (h - mu) / jnp.sqrt(var + 1e-5) * g_ref[...] + b_ref[...]
        h = h * jax.nn.sigmoid(h)
        h = h + jax.lax.dot_general(px, Wp_ref[...], (((1,), (0,)), ((), ())),
                                    preferred_element_type=jnp.float32)
        hs[rows, :] = h
        d = (px[:, 6:7] > 0).astype(jnp.int32)
        a = (px[:, 7:8] > 0).astype(jnp.int32)
        ar = (sx[:, 1:2] > 0).astype(jnp.int32)
        fqs[rows, :] = d + 2 * a + 4 * ar
        dr = (pxT_ref[6:7, :] > 0).astype(jnp.int32)
        ac = (pxT_ref[7:8, :] > 0).astype(jnp.int32)
        arr = (sxT_ref[1:2, :] > 0).astype(jnp.int32)
        gks[...] = 2 * dr + ac + 4 * arr
        mask = ((fqs[rows, :] & gks[...]) != 0) & (nc_ref[...] != nr_ref[...])
        anyf = jnp.any(mask, axis=1, keepdims=True).astype(jnp.float32)
        bias_s[rows, :] = jnp.where(
            mask, 0.0, -30000.0 * anyf).astype(jnp.bfloat16)
        hb = h.astype(jnp.bfloat16)
        qs[rows, :] = (jax.lax.dot_general(
            hb, Wq_ref[...], (((1,), (0,)), ((), ())),
            preferred_element_type=jnp.float32)
            * (_SCALE * anyf)).astype(jnp.bfloat16)
        kblk = jax.lax.dot_general(
            hb, Wk_ref[...], (((1,), (0,)), ((), ())),
            preferred_element_type=jnp.float32).astype(jnp.bfloat16)
        ks[:, rows] = kblk.T
        vblk = jax.lax.dot_general(
            hb, Wv_ref[...], (((1,), (0,)), ((), ())),
            preferred_element_type=jnp.float32).astype(jnp.bfloat16)
        ones = jnp.ones((_BA, 1), jnp.bfloat16)
        zpad = jnp.zeros((_BA, 128 - _DH - 1), jnp.bfloat16)
        for hh in range(_NH):
            sl = slice(hh * _DH, (hh + 1) * _DH)
            vs[rows, hh * 128:(hh + 1) * 128] = jnp.concatenate(
                [vblk[:, sl], ones, zpad], axis=1)

    @pl.when(i >= _NP)
    def _attn():
        j = i - _NP
        rows = pl.ds(j * _BQ, _BQ)
        q = qs[rows, :]
        v = vs[...]
        parts = []
        for hh in range(_NH):
            sl = slice(hh * _DH, (hh + 1) * _DH)
            s = jax.lax.dot_general(q[:, sl], ks[sl, :],
                                    (((1,), (0,)), ((), ())),
                                    preferred_element_type=jnp.float32)
            p = jnp.exp2(s + bias_s[rows, :].astype(jnp.float32)
                         ).astype(jnp.bfloat16)
            pv = jax.lax.dot_general(
                p, v[:, hh * 128:(hh + 1) * 128], (((1,), (0,)), ((), ())),
                preferred_element_type=jnp.float32)
            parts.append(pv[:, :_DH] / pv[:, _DH:_DH + 1])
        ctx_all = jnp.concatenate(parts, axis=1).astype(jnp.bfloat16)
        o_ref[...] = hs[rows, :] + jax.lax.dot_general(
            ctx_all, Wo_ref[...], (((1,), (0,)), ((), ())),
            preferred_element_type=jnp.float32)


def kernel(physics_x, structural_x, W_struct, b_struct, gamma, beta,
           W_phys, Wq, Wk, Wv, Wo, atom_to_nuc):
    nuc_col = atom_to_nuc.astype(jnp.int32).reshape(_N, 1)
    nuc_row = atom_to_nuc.astype(jnp.int32).reshape(1, _N)

    def _lo(i):
        return jnp.minimum(i, _NP - 1)

    def _hi(i):
        return jnp.maximum(i - _NP, 0)

    out = pl.pallas_call(
        _body,
        grid=(_NP + _NQ,),
        in_specs=[
            pl.BlockSpec((_BA, 10), lambda i: (_lo(i), 0)),
            pl.BlockSpec((_BA, 4), lambda i: (_lo(i), 0)),
            pl.BlockSpec((10, _N), lambda i: (0, 0)),
            pl.BlockSpec((4, _N), lambda i: (0, 0)),
            pl.BlockSpec((4, _H), lambda i: (0, 0)),
            pl.BlockSpec((1, _H), lambda i: (0, 0)),
            pl.BlockSpec((1, _H), lambda i: (0, 0)),
            pl.BlockSpec((1, _H), lambda i: (0, 0)),
            pl.BlockSpec((10, _H), lambda i: (0, 0)),
            pl.BlockSpec((_H, _H), lambda i: (0, 0)),
            pl.BlockSpec((_H, _H), lambda i: (0, 0)),
            pl.BlockSpec((_H, _H), lambda i: (0, 0)),
            pl.BlockSpec((_H, _H), lambda i: (0, 0)),
            pl.BlockSpec((_BA, 1), lambda i: (_lo(i), 0)),
            pl.BlockSpec((1, _N), lambda i: (0, 0)),
        ],
        out_specs=pl.BlockSpec((_BQ, _H), lambda i: (_hi(i), 0)),
        out_shape=jax.ShapeDtypeStruct((_N, _H), jnp.float32),
        scratch_shapes=[
            pltpu.VMEM((_N, _H), jnp.float32),
            pltpu.VMEM((_N, _H), jnp.bfloat16),
            pltpu.VMEM((_H, _N), jnp.bfloat16),
            pltpu.VMEM((_N, _NH * 128), jnp.bfloat16),
            pltpu.VMEM((_N, _N), jnp.bfloat16),
            pltpu.VMEM((_N, 1), jnp.int32),
            pltpu.VMEM((1, _N), jnp.int32),
        ],
    )(physics_x, structural_x, physics_x.T, structural_x.T,
      W_struct, b_struct.reshape(1, _H), gamma.reshape(1, _H),
      beta.reshape(1, _H), W_phys, Wq.astype(jnp.bfloat16),
      Wk.astype(jnp.bfloat16), Wv.astype(jnp.bfloat16),
      Wo.astype(jnp.bfloat16), nuc_col, nuc_row)
    return out


# speedup vs baseline: 1.1498x; 1.0340x over previous
import jax
import jax.numpy as jnp
from jax.experimental import pallas as pl
from jax.experimental.pallas import tpu as pltpu

_N, _H, _NH, _DH = 2048, 512, 8, 64
_BA = 512
_BQ = 1024
_NP = _N // _BA
_NQ = _N // _BQ
_SCALE = 0.125 * 1.4426950408889634


def _body(px_ref, sx_ref, pxT_ref, sxT_ref, Ws_ref, bs_ref, g_ref, b_ref,
          Wp_ref, Wq_ref, Wk_ref, Wv_ref, Wo_ref, nc_ref, nr_ref,
          o_ref,
          hs, qs, ks, vs, bias_s, fqs, gks):
    i = pl.program_id(0)

    @pl.when(i < _NP)
    def _embed():
        rows = pl.ds(i * _BA, _BA)
        px = px_ref[...]
        sx = sx_ref[...]
        h = jax.lax.dot_general(sx, Ws_ref[...], (((1,), (0,)), ((), ())),
                                preferred_element_type=jnp.float32)
        h = h + bs_ref[...]
        mu = jnp.mean(h, axis=1, keepdims=True)
        var = jnp.mean((h - mu) ** 2, axis=1, keepdims=True)
        h = (h - mu) / jnp.sqrt(var + 1e-5) * g_ref[...] + b_ref[...]
        h = h * jax.nn.sigmoid(h)
        h = h + jax.lax.dot_general(px, Wp_ref[...], (((1,), (0,)), ((), ())),
                                    preferred_element_type=jnp.float32)
        hs[rows, :] = h
        d = (px[:, 6:7] > 0).astype(jnp.int32)
        a = (px[:, 7:8] > 0).astype(jnp.int32)
        ar = (sx[:, 1:2] > 0).astype(jnp.int32)
        fqs[rows, :] = d + 2 * a + 4 * ar
        dr = (pxT_ref[6:7, :] > 0).astype(jnp.int32)
        ac = (pxT_ref[7:8, :] > 0).astype(jnp.int32)
        arr = (sxT_ref[1:2, :] > 0).astype(jnp.int32)
        gks[...] = 2 * dr + ac + 4 * arr
        mask = ((fqs[rows, :] & gks[...]) != 0) & (nc_ref[...] != nr_ref[...])
        anyf = jnp.any(mask, axis=1, keepdims=True).astype(jnp.float32)
        bias_s[rows, :] = jnp.where(
            mask, 0.0, -30000.0 * anyf).astype(jnp.bfloat16)
        hb = h.astype(jnp.bfloat16)
        qs[rows, :] = (jax.lax.dot_general(
            hb, Wq_ref[...], (((1,), (0,)), ((), ())),
            preferred_element_type=jnp.float32)
            * (_SCALE * anyf)).astype(jnp.bfloat16)
        kblk = jax.lax.dot_general(
            hb, Wk_ref[...], (((1,), (0,)), ((), ())),
            preferred_element_type=jnp.float32).astype(jnp.bfloat16)
        ks[:, rows] = kblk.T
        vblk = jax.lax.dot_general(
            hb, Wv_ref[...], (((1,), (0,)), ((), ())),
            preferred_element_type=jnp.float32).astype(jnp.bfloat16)
        ones = jnp.ones((_BA, 1), jnp.bfloat16)
        zpad = jnp.zeros((_BA, 128 - _DH - 1), jnp.bfloat16)
        for hh in range(_NH):
            sl = slice(hh * _DH, (hh + 1) * _DH)
            vs[rows, hh * 128:(hh + 1) * 128] = jnp.concatenate(
                [vblk[:, sl], ones, zpad], axis=1)

    @pl.when(i >= _NP)
    def _attn():
        j = i - _NP
        rows = pl.ds(j * _BQ, _BQ)
        q = qs[rows, :]
        v = vs[...]
        parts = []
        for hh in range(_NH):
            sl = slice(hh * _DH, (hh + 1) * _DH)
            s = jax.lax.dot_general(q[:, sl], ks[sl, :],
                                    (((1,), (0,)), ((), ())),
                                    preferred_element_type=jnp.float32)
            p = jnp.exp2(s + bias_s[rows, :].astype(jnp.float32)
                         ).astype(jnp.bfloat16)
            pv = jax.lax.dot_general(
                p, v[:, hh * 128:(hh + 1) * 128], (((1,), (0,)), ((), ())),
                preferred_element_type=jnp.float32)
            parts.append(pv[:, :_DH] / pv[:, _DH:_DH + 1])
        ctx_all = jnp.concatenate(parts, axis=1).astype(jnp.bfloat16)
        o_ref[...] = hs[rows, :] + jax.lax.dot_general(
            ctx_all, Wo_ref[...], (((1,), (0,)), ((), ())),
            preferred_element_type=jnp.float32)


def kernel(physics_x, structural_x, W_struct, b_struct, gamma, beta,
           W_phys, Wq, Wk, Wv, Wo, atom_to_nuc):
    nuc_col = atom_to_nuc.astype(jnp.int32).reshape(_N, 1)
    nuc_row = atom_to_nuc.astype(jnp.int32).reshape(1, _N)

    def _lo(i):
        return jnp.minimum(i, _NP - 1)

    def _hi(i):
        return jnp.maximum(i - _NP, 0)

    out = pl.pallas_call(
        _body,
        grid=(_NP + _NQ,),
        in_specs=[
            pl.BlockSpec((_BA, 10), lambda i: (_lo(i), 0)),
            pl.BlockSpec((_BA, 4), lambda i: (_lo(i), 0)),
            pl.BlockSpec((10, _N), lambda i: (0, 0)),
            pl.BlockSpec((4, _N), lambda i: (0, 0)),
            pl.BlockSpec((4, _H), lambda i: (0, 0)),
            pl.BlockSpec((1, _H), lambda i: (0, 0)),
            pl.BlockSpec((1, _H), lambda i: (0, 0)),
            pl.BlockSpec((1, _H), lambda i: (0, 0)),
            pl.BlockSpec((10, _H), lambda i: (0, 0)),
            pl.BlockSpec((_H, _H), lambda i: (0, 0)),
            pl.BlockSpec((_H, _H), lambda i: (0, 0)),
            pl.BlockSpec((_H, _H), lambda i: (0, 0)),
            pl.BlockSpec((_H, _H), lambda i: (0, 0)),
            pl.BlockSpec((_BA, 1), lambda i: (_lo(i), 0)),
            pl.BlockSpec((1, _N), lambda i: (0, 0)),
        ],
        out_specs=pl.BlockSpec((_BQ, _H), lambda i: (_hi(i), 0)),
        out_shape=jax.ShapeDtypeStruct((_N, _H), jnp.float32),
        scratch_shapes=[
            pltpu.VMEM((_N, _H), jnp.float32),
            pltpu.VMEM((_N, _H), jnp.bfloat16),
            pltpu.VMEM((_H, _N), jnp.bfloat16),
            pltpu.VMEM((_N, _NH * 128), jnp.bfloat16),
            pltpu.VMEM((_N, _N), jnp.bfloat16),
            pltpu.VMEM((_N, 1), jnp.int32),
            pltpu.VMEM((1, _N), jnp.int32),
        ],
    )(physics_x, structural_x, physics_x.T, structural_x.T,
      W_struct, b_struct.reshape(1, _H), gamma.reshape(1, _H),
      beta.reshape(1, _H), W_phys, Wq.astype(jnp.bfloat16),
      Wk.astype(jnp.bfloat16), Wv.astype(jnp.bfloat16),
      Wo.astype(jnp.bfloat16), nuc_col, nuc_row)
    return out


# speedup vs baseline: 1.1647x; 1.0129x over previous
import jax
import jax.numpy as jnp
from jax.experimental import pallas as pl
from jax.experimental.pallas import tpu as pltpu

_N, _H, _NH, _DH = 2048, 512, 8, 64
_BA = 1024
_BQ = 1024
_NP = _N // _BA
_NQ = _N // _BQ
_SCALE = 0.125 * 1.4426950408889634


def _body(px_ref, sx_ref, pxT_ref, sxT_ref, Ws_ref, bs_ref, g_ref, b_ref,
          Wp_ref, Wq_ref, Wk_ref, Wv_ref, Wo_ref, nc_ref, nr_ref,
          o_ref,
          hs, qs, ks, vs, bias_s, fqs, gks):
    i = pl.program_id(0)

    @pl.when(i < _NP)
    def _embed():
        rows = pl.ds(i * _BA, _BA)
        px = px_ref[...]
        sx = sx_ref[...]
        h = jax.lax.dot_general(sx, Ws_ref[...], (((1,), (0,)), ((), ())),
                                preferred_element_type=jnp.float32)
        h = h + bs_ref[...]
        mu = jnp.mean(h, axis=1, keepdims=True)
        var = jnp.mean((h - mu) ** 2, axis=1, keepdims=True)
        h = (h - mu) / jnp.sqrt(var + 1e-5) * g_ref[...] + b_ref[...]
        h = h * jax.nn.sigmoid(h)
        h = h + jax.lax.dot_general(px, Wp_ref[...], (((1,), (0,)), ((), ())),
                                    preferred_element_type=jnp.float32)
        hs[rows, :] = h
        d = (px[:, 6:7] > 0).astype(jnp.int32)
        a = (px[:, 7:8] > 0).astype(jnp.int32)
        ar = (sx[:, 1:2] > 0).astype(jnp.int32)
        fqs[rows, :] = d + 2 * a + 4 * ar
        dr = (pxT_ref[6:7, :] > 0).astype(jnp.int32)
        ac = (pxT_ref[7:8, :] > 0).astype(jnp.int32)
        arr = (sxT_ref[1:2, :] > 0).astype(jnp.int32)
        gks[...] = 2 * dr + ac + 4 * arr
        mask = ((fqs[rows, :] & gks[...]) != 0) & (nc_ref[...] != nr_ref[...])
        anyf = jnp.any(mask, axis=1, keepdims=True).astype(jnp.float32)
        bias_s[rows, :] = jnp.where(
            mask, 0.0, -30000.0 * anyf).astype(jnp.bfloat16)
        hb = h.astype(jnp.bfloat16)
        qs[rows, :] = (jax.lax.dot_general(
            hb, Wq_ref[...], (((1,), (0,)), ((), ())),
            preferred_element_type=jnp.float32)
            * (_SCALE * anyf)).astype(jnp.bfloat16)
        kblk = jax.lax.dot_general(
            hb, Wk_ref[...], (((1,), (0,)), ((), ())),
            preferred_element_type=jnp.float32).astype(jnp.bfloat16)
        ks[:, rows] = kblk.T
        vblk = jax.lax.dot_general(
            hb, Wv_ref[...], (((1,), (0,)), ((), ())),
            preferred_element_type=jnp.float32).astype(jnp.bfloat16)
        ones = jnp.ones((_BA, 1), jnp.bfloat16)
        zpad = jnp.zeros((_BA, 128 - _DH - 1), jnp.bfloat16)
        for hh in range(_NH):
            sl = slice(hh * _DH, (hh + 1) * _DH)
            vs[rows, hh * 128:(hh + 1) * 128] = jnp.concatenate(
                [vblk[:, sl], ones, zpad], axis=1)

    @pl.when(i >= _NP)
    def _attn():
        j = i - _NP
        rows = pl.ds(j * _BQ, _BQ)
        q = qs[rows, :]
        v = vs[...]
        parts = []
        for hh in range(_NH):
            sl = slice(hh * _DH, (hh + 1) * _DH)
            s = jax.lax.dot_general(q[:, sl], ks[sl, :],
                                    (((1,), (0,)), ((), ())),
                                    preferred_element_type=jnp.float32)
            p = jnp.exp2(s + bias_s[rows, :].astype(jnp.float32)
                         ).astype(jnp.bfloat16)
            pv = jax.lax.dot_general(
                p, v[:, hh * 128:(hh + 1) * 128], (((1,), (0,)), ((), ())),
                preferred_element_type=jnp.float32)
            parts.append(pv[:, :_DH] / pv[:, _DH:_DH + 1])
        ctx_all = jnp.concatenate(parts, axis=1).astype(jnp.bfloat16)
        o_ref[...] = hs[rows, :] + jax.lax.dot_general(
            ctx_all, Wo_ref[...], (((1,), (0,)), ((), ())),
            preferred_element_type=jnp.float32)


def kernel(physics_x, structural_x, W_struct, b_struct, gamma, beta,
           W_phys, Wq, Wk, Wv, Wo, atom_to_nuc):
    nuc_col = atom_to_nuc.astype(jnp.int32).reshape(_N, 1)
    nuc_row = atom_to_nuc.astype(jnp.int32).reshape(1, _N)

    def _lo(i):
        return jnp.minimum(i, _NP - 1)

    def _hi(i):
        return jnp.maximum(i - _NP, 0)

    out = pl.pallas_call(
        _body,
        grid=(_NP + _NQ,),
        in_specs=[
            pl.BlockSpec((_BA, 10), lambda i: (_lo(i), 0)),
            pl.BlockSpec((_BA, 4), lambda i: (_lo(i), 0)),
            pl.BlockSpec((10, _N), lambda i: (0, 0)),
            pl.BlockSpec((4, _N), lambda i: (0, 0)),
            pl.BlockSpec((4, _H), lambda i: (0, 0)),
            pl.BlockSpec((1, _H), lambda i: (0, 0)),
            pl.BlockSpec((1, _H), lambda i: (0, 0)),
            pl.BlockSpec((1, _H), lambda i: (0, 0)),
            pl.BlockSpec((10, _H), lambda i: (0, 0)),
            pl.BlockSpec((_H, _H), lambda i: (0, 0)),
            pl.BlockSpec((_H, _H), lambda i: (0, 0)),
            pl.BlockSpec((_H, _H), lambda i: (0, 0)),
            pl.BlockSpec((_H, _H), lambda i: (0, 0)),
            pl.BlockSpec((_BA, 1), lambda i: (_lo(i), 0)),
            pl.BlockSpec((1, _N), lambda i: (0, 0)),
        ],
        out_specs=pl.BlockSpec((_BQ, _H), lambda i: (_hi(i), 0)),
        out_shape=jax.ShapeDtypeStruct((_N, _H), jnp.float32),
        scratch_shapes=[
            pltpu.VMEM((_N, _H), jnp.float32),
            pltpu.VMEM((_N, _H), jnp.bfloat16),
            pltpu.VMEM((_H, _N), jnp.bfloat16),
            pltpu.VMEM((_N, _NH * 128), jnp.bfloat16),
            pltpu.VMEM((_N, _N), jnp.bfloat16),
            pltpu.VMEM((_N, 1), jnp.int32),
            pltpu.VMEM((1, _N), jnp.int32),
        ],
    )(physics_x, structural_x, physics_x.T, structural_x.T,
      W_struct, b_struct.reshape(1, _H), gamma.reshape(1, _H),
      beta.reshape(1, _H), W_phys, Wq.astype(jnp.bfloat16),
      Wk.astype(jnp.bfloat16), Wv.astype(jnp.bfloat16),
      Wo.astype(jnp.bfloat16), nuc_col, nuc_row)
    return out
